# baseline (speedup 1.0000x reference)
import jax
import jax.numpy as jnp
from jax.experimental import pallas as pl
from jax.experimental.pallas import tpu as pltpu

_NQ = 8


def _pos_body(col_ref, row_ref, out_hbm, slab, sems):
    h, w = slab.shape[0], slab.shape[1]
    d = col_ref.shape[1]
    b = out_hbm.shape[0]
    hq = h // _NQ
    col_img = jnp.broadcast_to(col_ref[...][None, :, :], (hq, w, d))
    copies = []
    for q in range(_NQ):
        row_img = jnp.broadcast_to(
            row_ref[pl.ds(q * hq, hq), :][:, None, :], (hq, w, d)
        )
        slab[pl.ds(q * hq, hq)] = jnp.concatenate([col_img, row_img], axis=-1)
        for i in range(b):
            cp = pltpu.make_async_copy(
                slab.at[pl.ds(q * hq, hq)],
                out_hbm.at[i, pl.ds(q * hq, hq)],
                sems.at[i, q],
            )
            cp.start()
            copies.append(cp)
    for cp in copies:
        cp.wait()


def kernel(x, row_embed, col_embed):
    b = x.shape[0]
    h, w = x.shape[-2], x.shape[-1]
    d = col_embed.shape[1]
    out = pl.pallas_call(
        _pos_body,
        grid=(1,),
        in_specs=[
            pl.BlockSpec((w, d), lambda i: (0, 0)),
            pl.BlockSpec((h, d), lambda i: (0, 0)),
        ],
        out_specs=pl.BlockSpec(memory_space=pltpu.HBM),
        out_shape=jax.ShapeDtypeStruct((b, h, w, 2 * d), jnp.float32),
        scratch_shapes=[
            pltpu.VMEM((h, w, 2 * d), jnp.float32),
            pltpu.SemaphoreType.DMA((b, _NQ)),
        ],
    )(col_embed, row_embed)
    return jnp.transpose(out, (0, 3, 1, 2))


# speedup vs baseline: 1.0544x; 1.0544x over previous
import jax
import jax.numpy as jnp
from jax.experimental import pallas as pl
from jax.experimental.pallas import tpu as pltpu

_NQ = 8


def _pos_body(col_ref, row_ref, out_hbm, slab, sems):
    h, w = slab.shape[0], slab.shape[1]
    d = col_ref.shape[1]
    b = out_hbm.shape[0]
    hq = h // _NQ
    col_img = jnp.broadcast_to(col_ref[...][None, :, :], (hq, w, d))
    copies = []
    for q in range(_NQ):
        row_img = jnp.broadcast_to(
            row_ref[pl.ds(q * hq, hq), :][:, None, :], (hq, w, d)
        )
        slab[pl.ds(q * hq, hq), :, :d] = col_img
        slab[pl.ds(q * hq, hq), :, d:] = row_img
        for i in range(b):
            cp = pltpu.make_async_copy(
                slab.at[pl.ds(q * hq, hq)],
                out_hbm.at[i, pl.ds(q * hq, hq)],
                sems.at[i],
            )
            cp.start()
            copies.append(cp)
    for cp in copies:
        cp.wait()


def kernel(x, row_embed, col_embed):
    b = x.shape[0]
    h, w = x.shape[-2], x.shape[-1]
    d = col_embed.shape[1]
    out = pl.pallas_call(
        _pos_body,
        grid=(1,),
        in_specs=[
            pl.BlockSpec((w, d), lambda i: (0, 0)),
            pl.BlockSpec((h, d), lambda i: (0, 0)),
        ],
        out_specs=pl.BlockSpec(memory_space=pltpu.HBM),
        out_shape=jax.ShapeDtypeStruct((b, h, w, 2 * d), jnp.float32),
        scratch_shapes=[
            pltpu.VMEM((h, w, 2 * d), jnp.float32),
            pltpu.SemaphoreType.DMA((b,)),
        ],
    )(col_embed, row_embed)
    return jnp.transpose(out, (0, 3, 1, 2))
